# baseline (device time: 94564 ns/iter reference)
import jax
import jax.numpy as jnp
from jax import lax
from jax.experimental import pallas as pl
from jax.experimental.pallas import tpu as pltpu

ZN = 4
PN = 8
N_DEV = ZN * PN


def kernel(x):
    m, n = x.shape
    slice_r = m // PN
    chunk_r = slice_r // ZN
    half = n // 2
    MESH = pl.DeviceIdType.MESH

    def body(x_ref, out_ref,
             p1u_buf, p1d_buf, z2u_buf, z2d_buf,
             z3u_buf, z3d_buf, p4u_buf, p4d_buf,
             p1u_ss, p1u_rs, p1d_ss, p1d_rs,
             z2u_ss, z2u_rs, z2d_ss, z2d_rs,
             z3u_ss, z3u_rs, z3d_ss, z3d_rs,
             p4u_ss, p4u_rs, p4d_ss, p4d_rs):
        me = lax.axis_index("i")
        zi = me // PN
        p = me % PN

        y_ = p // 2
        x_ = jnp.where(y_ % 2 == 0, p % 2, 1 - p % 2)
        q = jnp.where(x_ == 0, y_, 2 * ZN - 1 - y_)

        def p_of_q(qq):
            qq = qq % PN
            yy = jnp.where(qq < ZN, qq, PN - 1 - qq)
            xx = jnp.where(qq < ZN, 0, 1)
            return 2 * yy + jnp.where(yy % 2 == 0, xx, 1 - xx)

        z_up = ((zi + 1) % ZN) * PN + p
        z_dn = ((zi - 1) % ZN) * PN + p
        pl_r = zi * PN + p_of_q(q + 1)
        pl_l = zi * PN + p_of_q(q - 1)

        U = pl.ds(0, half)
        D = pl.ds(half, half)

        def slice_rows(idx):
            return pl.ds((idx % PN) * slice_r, slice_r)

        def chunk_rows(slice_idx, cidx):
            return pl.ds((slice_idx % PN) * slice_r
                         + (cidx % ZN) * chunk_r, chunk_r)

        def rcopy(src_ref, dst_ref, ss, rs, dev):
            return pltpu.make_async_remote_copy(
                src_ref=src_ref, dst_ref=dst_ref, send_sem=ss, recv_sem=rs,
                device_id=(dev,), device_id_type=MESH)

        barrier_sem = pltpu.get_barrier_semaphore()
        for nbr in (z_up, z_dn, pl_r, pl_l):
            pl.semaphore_signal(barrier_sem, inc=1,
                                device_id=(nbr,), device_id_type=MESH)
        pl.semaphore_wait(barrier_sem, 4)

        out_ref[:, :] = x_ref[:, :]

        def p1_r(t):
            return rcopy(out_ref.at[slice_rows(q - t), U], p1u_buf.at[t],
                         p1u_ss.at[t], p1u_rs.at[t], pl_r)

        def p1_l(t):
            return rcopy(out_ref.at[slice_rows(q + 2 + t), D], p1d_buf.at[t],
                         p1d_ss.at[t], p1d_rs.at[t], pl_l)

        p1 = [p1_r(0), p1_l(0)]
        p1[0].start()
        p1[1].start()
        for t in range(PN - 1):
            p1[2 * t].wait_recv()
            ru = slice_rows(q - t - 1)
            out_ref[ru, U] = out_ref[ru, U] + p1u_buf[t]
            if t < PN - 2:
                nxt = p1_r(t + 1)
                nxt.start()
                p1.append(nxt)
            p1[2 * t + 1].wait_recv()
            rd = slice_rows(q + 3 + t)
            out_ref[rd, D] = out_ref[rd, D] + p1d_buf[t]
            if t < PN - 2:
                nxt = p1_l(t + 1)
                nxt.start()
                p1.append(nxt)
        for r in p1:
            r.wait_send()

        B = (q + 1) % PN

        def p2_u(s):
            return rcopy(out_ref.at[chunk_rows(B, zi - s), U], z2u_buf.at[s],
                         z2u_ss.at[s], z2u_rs.at[s], z_up)

        def p2_d(s):
            return rcopy(out_ref.at[chunk_rows(B, zi - 2 + s), D],
                         z2d_buf.at[s],
                         z2d_ss.at[s], z2d_rs.at[s], z_dn)

        p2 = [p2_u(0), p2_d(0)]
        p2[0].start()
        p2[1].start()
        for s in range(ZN - 1):
            p2[2 * s].wait_recv()
            ru = chunk_rows(B, zi - s - 1)
            out_ref[ru, U] = out_ref[ru, U] + z2u_buf[s]
            if s < ZN - 2:
                nxt = p2_u(s + 1)
                nxt.start()
                p2.append(nxt)
            p2[2 * s + 1].wait_recv()
            rd = chunk_rows(B, zi - 1 + s)
            out_ref[rd, D] = out_ref[rd, D] + z2d_buf[s]
            if s < ZN - 2:
                nxt = p2_d(s + 1)
                nxt.start()
                p2.append(nxt)
        for r in p2:
            r.wait_send()

        F = (zi + 1) % ZN

        p3 = []
        for u in range(ZN - 1):
            src_u = (out_ref.at[chunk_rows(B, F), U] if u == 0
                     else z3u_buf.at[u - 1])
            src_d = (out_ref.at[chunk_rows(B, F), D] if u == 0
                     else z3d_buf.at[u - 1])
            up = rcopy(src_u, z3u_buf.at[u], z3u_ss.at[u], z3u_rs.at[u], z_up)
            dn = rcopy(src_d, z3d_buf.at[u], z3d_ss.at[u], z3d_rs.at[u], z_dn)
            up.start()
            dn.start()
            p3 += [up, dn]
            up.wait_recv()
            out_ref[chunk_rows(B, zi - u), U] = z3u_buf[u]
            dn.wait_recv()
            out_ref[chunk_rows(B, zi + 2 + u), D] = z3d_buf[u]
        for r in p3:
            r.wait_send()


        p4 = []
        for t in range(PN - 1):
            src_u = (out_ref.at[slice_rows(B), U] if t == 0
                     else p4u_buf.at[t - 1])
            src_d = (out_ref.at[slice_rows(B), D] if t == 0
                     else p4d_buf.at[t - 1])
            rr = rcopy(src_u, p4u_buf.at[t], p4u_ss.at[t], p4u_rs.at[t], pl_r)
            ll = rcopy(src_d, p4d_buf.at[t], p4d_ss.at[t], p4d_rs.at[t], pl_l)
            rr.start()
            ll.start()
            p4 += [rr, ll]
            rr.wait_recv()
            out_ref[slice_rows(q - t), U] = p4u_buf[t]
            ll.wait_recv()
            out_ref[slice_rows(q + 2 + t), D] = p4d_buf[t]
        for r in p4:
            r.wait_send()

    return pl.pallas_call(
        body,
        out_shape=jax.ShapeDtypeStruct((m, n), x.dtype),
        in_specs=[pl.BlockSpec(memory_space=pltpu.VMEM)],
        out_specs=pl.BlockSpec(memory_space=pltpu.VMEM),
        scratch_shapes=[
            pltpu.VMEM((PN - 1, slice_r, half), x.dtype),
            pltpu.VMEM((PN - 1, slice_r, half), x.dtype),
            pltpu.VMEM((ZN - 1, chunk_r, half), x.dtype),
            pltpu.VMEM((ZN - 1, chunk_r, half), x.dtype),
            pltpu.VMEM((ZN - 1, chunk_r, half), x.dtype),
            pltpu.VMEM((ZN - 1, chunk_r, half), x.dtype),
            pltpu.VMEM((PN - 1, slice_r, half), x.dtype),
            pltpu.VMEM((PN - 1, slice_r, half), x.dtype),
            pltpu.SemaphoreType.DMA((PN - 1,)),
            pltpu.SemaphoreType.DMA((PN - 1,)),
            pltpu.SemaphoreType.DMA((PN - 1,)),
            pltpu.SemaphoreType.DMA((PN - 1,)),
            pltpu.SemaphoreType.DMA((ZN - 1,)),
            pltpu.SemaphoreType.DMA((ZN - 1,)),
            pltpu.SemaphoreType.DMA((ZN - 1,)),
            pltpu.SemaphoreType.DMA((ZN - 1,)),
            pltpu.SemaphoreType.DMA((ZN - 1,)),
            pltpu.SemaphoreType.DMA((ZN - 1,)),
            pltpu.SemaphoreType.DMA((ZN - 1,)),
            pltpu.SemaphoreType.DMA((ZN - 1,)),
            pltpu.SemaphoreType.DMA((PN - 1,)),
            pltpu.SemaphoreType.DMA((PN - 1,)),
            pltpu.SemaphoreType.DMA((PN - 1,)),
            pltpu.SemaphoreType.DMA((PN - 1,)),
        ],
        compiler_params=pltpu.CompilerParams(collective_id=0),
    )(x)


# device time: 94331 ns/iter; 1.0025x vs baseline; 1.0025x over previous
import jax
import jax.numpy as jnp
from jax import lax
from jax.experimental import pallas as pl
from jax.experimental.pallas import tpu as pltpu

ZN = 4
PN = 8
N_DEV = ZN * PN


def kernel(x):
    m, n = x.shape
    slice_r = m // PN
    chunk_r = slice_r // ZN
    half = n // 2
    MESH = pl.DeviceIdType.MESH

    def body(x_ref, out_ref,
             p1u_buf, p1d_buf, z2u_buf, z2d_buf,
             z3u_buf, z3d_buf, p4u_buf, p4d_buf,
             p1u_ss, p1u_rs, p1d_ss, p1d_rs,
             z2u_ss, z2u_rs, z2d_ss, z2d_rs,
             z3u_ss, z3u_rs, z3d_ss, z3d_rs,
             p4u_ss, p4u_rs, p4d_ss, p4d_rs):
        me = lax.axis_index("i")
        zi = me // PN
        p = me % PN

        y_ = p // 2
        x_ = jnp.where(y_ % 2 == 0, p % 2, 1 - p % 2)
        q = jnp.where(x_ == 0, y_, 2 * ZN - 1 - y_)

        def p_of_q(qq):
            qq = qq % PN
            yy = jnp.where(qq < ZN, qq, PN - 1 - qq)
            xx = jnp.where(qq < ZN, 0, 1)
            return 2 * yy + jnp.where(yy % 2 == 0, xx, 1 - xx)

        z_up = ((zi + 1) % ZN) * PN + p
        z_dn = ((zi - 1) % ZN) * PN + p
        pl_r = zi * PN + p_of_q(q + 1)
        pl_l = zi * PN + p_of_q(q - 1)

        U = pl.ds(0, half)
        D = pl.ds(half, half)

        def slice_rows(idx):
            return pl.ds((idx % PN) * slice_r, slice_r)

        def chunk_rows(slice_idx, cidx):
            return pl.ds((slice_idx % PN) * slice_r
                         + (cidx % ZN) * chunk_r, chunk_r)

        def rcopy(src_ref, dst_ref, ss, rs, dev):
            return pltpu.make_async_remote_copy(
                src_ref=src_ref, dst_ref=dst_ref, send_sem=ss, recv_sem=rs,
                device_id=(dev,), device_id_type=MESH)

        barrier_sem = pltpu.get_barrier_semaphore()
        for nbr in (z_up, z_dn, pl_r, pl_l):
            pl.semaphore_signal(barrier_sem, inc=1,
                                device_id=(nbr,), device_id_type=MESH)
        pl.semaphore_wait(barrier_sem, 4)

        def p1_r(t):
            src = (x_ref if t == 0 else out_ref).at[slice_rows(q - t), U]
            return rcopy(src, p1u_buf.at[t],
                         p1u_ss.at[t], p1u_rs.at[t], pl_r)

        def p1_l(t):
            src = (x_ref if t == 0 else out_ref).at[slice_rows(q + 2 + t), D]
            return rcopy(src, p1d_buf.at[t],
                         p1d_ss.at[t], p1d_rs.at[t], pl_l)

        p1 = [p1_r(0), p1_l(0)]
        p1[0].start()
        p1[1].start()

        out_ref[:, :] = x_ref[:, :]
        for t in range(PN - 1):
            p1[2 * t].wait_recv()
            ru = slice_rows(q - t - 1)
            out_ref[ru, U] = out_ref[ru, U] + p1u_buf[t]
            if t < PN - 2:
                nxt = p1_r(t + 1)
                nxt.start()
                p1.append(nxt)
            p1[2 * t + 1].wait_recv()
            rd = slice_rows(q + 3 + t)
            out_ref[rd, D] = out_ref[rd, D] + p1d_buf[t]
            if t < PN - 2:
                nxt = p1_l(t + 1)
                nxt.start()
                p1.append(nxt)
        for r in p1:
            r.wait_send()

        B = (q + 1) % PN

        def p2_u(s):
            return rcopy(out_ref.at[chunk_rows(B, zi - s), U], z2u_buf.at[s],
                         z2u_ss.at[s], z2u_rs.at[s], z_up)

        def p2_d(s):
            return rcopy(out_ref.at[chunk_rows(B, zi - 2 + s), D],
                         z2d_buf.at[s],
                         z2d_ss.at[s], z2d_rs.at[s], z_dn)

        p2 = [p2_u(0), p2_d(0)]
        p2[0].start()
        p2[1].start()
        for s in range(ZN - 1):
            p2[2 * s].wait_recv()
            ru = chunk_rows(B, zi - s - 1)
            out_ref[ru, U] = out_ref[ru, U] + z2u_buf[s]
            if s < ZN - 2:
                nxt = p2_u(s + 1)
                nxt.start()
                p2.append(nxt)
            p2[2 * s + 1].wait_recv()
            rd = chunk_rows(B, zi - 1 + s)
            out_ref[rd, D] = out_ref[rd, D] + z2d_buf[s]
            if s < ZN - 2:
                nxt = p2_d(s + 1)
                nxt.start()
                p2.append(nxt)
        for r in p2:
            r.wait_send()

        F = (zi + 1) % ZN

        p3 = []
        for u in range(ZN - 1):
            src_u = (out_ref.at[chunk_rows(B, F), U] if u == 0
                     else z3u_buf.at[u - 1])
            src_d = (out_ref.at[chunk_rows(B, F), D] if u == 0
                     else z3d_buf.at[u - 1])
            up = rcopy(src_u, z3u_buf.at[u], z3u_ss.at[u], z3u_rs.at[u], z_up)
            dn = rcopy(src_d, z3d_buf.at[u], z3d_ss.at[u], z3d_rs.at[u], z_dn)
            up.start()
            dn.start()
            p3 += [up, dn]
            up.wait_recv()
            out_ref[chunk_rows(B, zi - u), U] = z3u_buf[u]
            dn.wait_recv()
            out_ref[chunk_rows(B, zi + 2 + u), D] = z3d_buf[u]
        for r in p3:
            r.wait_send()


        p4 = []
        for t in range(PN - 1):
            src_u = (out_ref.at[slice_rows(B), U] if t == 0
                     else p4u_buf.at[t - 1])
            src_d = (out_ref.at[slice_rows(B), D] if t == 0
                     else p4d_buf.at[t - 1])
            rr = rcopy(src_u, p4u_buf.at[t], p4u_ss.at[t], p4u_rs.at[t], pl_r)
            ll = rcopy(src_d, p4d_buf.at[t], p4d_ss.at[t], p4d_rs.at[t], pl_l)
            rr.start()
            ll.start()
            p4 += [rr, ll]
            rr.wait_recv()
            out_ref[slice_rows(q - t), U] = p4u_buf[t]
            ll.wait_recv()
            out_ref[slice_rows(q + 2 + t), D] = p4d_buf[t]
        for r in p4:
            r.wait_send()

    return pl.pallas_call(
        body,
        out_shape=jax.ShapeDtypeStruct((m, n), x.dtype),
        in_specs=[pl.BlockSpec(memory_space=pltpu.VMEM)],
        out_specs=pl.BlockSpec(memory_space=pltpu.VMEM),
        scratch_shapes=[
            pltpu.VMEM((PN - 1, slice_r, half), x.dtype),
            pltpu.VMEM((PN - 1, slice_r, half), x.dtype),
            pltpu.VMEM((ZN - 1, chunk_r, half), x.dtype),
            pltpu.VMEM((ZN - 1, chunk_r, half), x.dtype),
            pltpu.VMEM((ZN - 1, chunk_r, half), x.dtype),
            pltpu.VMEM((ZN - 1, chunk_r, half), x.dtype),
            pltpu.VMEM((PN - 1, slice_r, half), x.dtype),
            pltpu.VMEM((PN - 1, slice_r, half), x.dtype),
            pltpu.SemaphoreType.DMA((PN - 1,)),
            pltpu.SemaphoreType.DMA((PN - 1,)),
            pltpu.SemaphoreType.DMA((PN - 1,)),
            pltpu.SemaphoreType.DMA((PN - 1,)),
            pltpu.SemaphoreType.DMA((ZN - 1,)),
            pltpu.SemaphoreType.DMA((ZN - 1,)),
            pltpu.SemaphoreType.DMA((ZN - 1,)),
            pltpu.SemaphoreType.DMA((ZN - 1,)),
            pltpu.SemaphoreType.DMA((ZN - 1,)),
            pltpu.SemaphoreType.DMA((ZN - 1,)),
            pltpu.SemaphoreType.DMA((ZN - 1,)),
            pltpu.SemaphoreType.DMA((ZN - 1,)),
            pltpu.SemaphoreType.DMA((PN - 1,)),
            pltpu.SemaphoreType.DMA((PN - 1,)),
            pltpu.SemaphoreType.DMA((PN - 1,)),
            pltpu.SemaphoreType.DMA((PN - 1,)),
        ],
        compiler_params=pltpu.CompilerParams(collective_id=0),
    )(x)


# device time: 92164 ns/iter; 1.0260x vs baseline; 1.0235x over previous
import jax
import jax.numpy as jnp
from jax import lax
from jax.experimental import pallas as pl
from jax.experimental.pallas import tpu as pltpu

ZN = 4
PN = 8
N_DEV = ZN * PN


def kernel(x):
    m, n = x.shape
    slice_r = m // PN
    chunk_r = slice_r // ZN
    half = n // 2
    MESH = pl.DeviceIdType.MESH

    def body(x_ref, out_ref,
             p1u_buf, p1d_buf, z2u_buf, z2d_buf,
             z3u_buf, z3d_buf, p4u_buf, p4d_buf,
             p1u_ss, p1u_rs, p1d_ss, p1d_rs,
             z2u_ss, z2u_rs, z2d_ss, z2d_rs,
             z3u_ss, z3u_rs, z3d_ss, z3d_rs,
             p4u_ss, p4u_rs, p4d_ss, p4d_rs):
        me = lax.axis_index("i")
        zi = me // PN
        p = me % PN

        y_ = p // 2
        x_ = jnp.where(y_ % 2 == 0, p % 2, 1 - p % 2)
        q = jnp.where(x_ == 0, y_, 2 * ZN - 1 - y_)

        def p_of_q(qq):
            qq = qq % PN
            yy = jnp.where(qq < ZN, qq, PN - 1 - qq)
            xx = jnp.where(qq < ZN, 0, 1)
            return 2 * yy + jnp.where(yy % 2 == 0, xx, 1 - xx)

        z_up = ((zi + 1) % ZN) * PN + p
        z_dn = ((zi - 1) % ZN) * PN + p
        pl_r = zi * PN + p_of_q(q + 1)
        pl_l = zi * PN + p_of_q(q - 1)

        U = pl.ds(0, half)
        D = pl.ds(half, half)

        def slice_rows(idx):
            return pl.ds((idx % PN) * slice_r, slice_r)

        def chunk_rows(slice_idx, cidx):
            return pl.ds((slice_idx % PN) * slice_r
                         + (cidx % ZN) * chunk_r, chunk_r)

        def rcopy(src_ref, dst_ref, ss, rs, dev):
            return pltpu.make_async_remote_copy(
                src_ref=src_ref, dst_ref=dst_ref, send_sem=ss, recv_sem=rs,
                device_id=(dev,), device_id_type=MESH)

        barrier_sem = pltpu.get_barrier_semaphore()
        for nbr in (z_up, z_dn, pl_r, pl_l):
            pl.semaphore_signal(barrier_sem, inc=1,
                                device_id=(nbr,), device_id_type=MESH)
        pl.semaphore_wait(barrier_sem, 4)

        def p1_r(t):
            src = (x_ref if t == 0 else out_ref).at[slice_rows(q - t), U]
            return rcopy(src, p1u_buf.at[t],
                         p1u_ss.at[t], p1u_rs.at[t], pl_r)

        def p1_l(t):
            src = (x_ref if t == 0 else out_ref).at[slice_rows(q + 2 + t), D]
            return rcopy(src, p1d_buf.at[t],
                         p1d_ss.at[t], p1d_rs.at[t], pl_l)

        p1 = [p1_r(0), p1_l(0)]
        p1[0].start()
        p1[1].start()

        out_ref[:, :] = x_ref[:, :]
        for t in range(PN - 1):
            p1[2 * t].wait_recv()
            ru = slice_rows(q - t - 1)
            out_ref[ru, U] = out_ref[ru, U] + p1u_buf[t]
            if t < PN - 2:
                nxt = p1_r(t + 1)
                nxt.start()
                p1.append(nxt)
            p1[2 * t + 1].wait_recv()
            rd = slice_rows(q + 3 + t)
            out_ref[rd, D] = out_ref[rd, D] + p1d_buf[t]
            if t < PN - 2:
                nxt = p1_l(t + 1)
                nxt.start()
                p1.append(nxt)
        for r in p1:
            r.wait_send()

        B = (q + 1) % PN

        def p2_u(s):
            return rcopy(out_ref.at[chunk_rows(B, zi - s), U], z2u_buf.at[s],
                         z2u_ss.at[s], z2u_rs.at[s], z_up)

        def p2_d(s):
            return rcopy(out_ref.at[chunk_rows(B, zi - 2 + s), D],
                         z2d_buf.at[s],
                         z2d_ss.at[s], z2d_rs.at[s], z_dn)

        p2 = [p2_u(0), p2_d(0)]
        p2[0].start()
        p2[1].start()
        for s in range(ZN - 1):
            p2[2 * s].wait_recv()
            ru = chunk_rows(B, zi - s - 1)
            out_ref[ru, U] = out_ref[ru, U] + z2u_buf[s]
            if s < ZN - 2:
                nxt = p2_u(s + 1)
                nxt.start()
                p2.append(nxt)
            p2[2 * s + 1].wait_recv()
            rd = chunk_rows(B, zi - 1 + s)
            out_ref[rd, D] = out_ref[rd, D] + z2d_buf[s]
            if s < ZN - 2:
                nxt = p2_d(s + 1)
                nxt.start()
                p2.append(nxt)
        for r in p2:
            r.wait_send()

        F = (zi + 1) % ZN

        def p3_u(u):
            src = (out_ref.at[chunk_rows(B, F), U] if u == 0
                   else z3u_buf.at[u - 1])
            return rcopy(src, z3u_buf.at[u], z3u_ss.at[u], z3u_rs.at[u], z_up)

        def p3_d(u):
            src = (out_ref.at[chunk_rows(B, F), D] if u == 0
                   else z3d_buf.at[u - 1])
            return rcopy(src, z3d_buf.at[u], z3d_ss.at[u], z3d_rs.at[u], z_dn)

        p3 = [p3_u(0), p3_d(0)]
        p3[0].start()
        p3[1].start()
        for u in range(ZN - 1):
            p3[2 * u].wait_recv()
            if u < ZN - 2:
                nxt = p3_u(u + 1)
                nxt.start()
                p3.append(nxt)
            p3[2 * u + 1].wait_recv()
            if u < ZN - 2:
                nxt = p3_d(u + 1)
                nxt.start()
                p3.append(nxt)
            out_ref[chunk_rows(B, zi - u), U] = z3u_buf[u]
            out_ref[chunk_rows(B, zi + 2 + u), D] = z3d_buf[u]
        for r in p3:
            r.wait_send()


        def p4_r(t):
            src = (out_ref.at[slice_rows(B), U] if t == 0
                   else p4u_buf.at[t - 1])
            return rcopy(src, p4u_buf.at[t], p4u_ss.at[t], p4u_rs.at[t], pl_r)

        def p4_l(t):
            src = (out_ref.at[slice_rows(B), D] if t == 0
                   else p4d_buf.at[t - 1])
            return rcopy(src, p4d_buf.at[t], p4d_ss.at[t], p4d_rs.at[t], pl_l)

        p4 = [p4_r(0), p4_l(0)]
        p4[0].start()
        p4[1].start()
        for t in range(PN - 1):
            p4[2 * t].wait_recv()
            if t < PN - 2:
                nxt = p4_r(t + 1)
                nxt.start()
                p4.append(nxt)
            p4[2 * t + 1].wait_recv()
            if t < PN - 2:
                nxt = p4_l(t + 1)
                nxt.start()
                p4.append(nxt)
            out_ref[slice_rows(q - t), U] = p4u_buf[t]
            out_ref[slice_rows(q + 2 + t), D] = p4d_buf[t]
        for r in p4:
            r.wait_send()

    return pl.pallas_call(
        body,
        out_shape=jax.ShapeDtypeStruct((m, n), x.dtype),
        in_specs=[pl.BlockSpec(memory_space=pltpu.VMEM)],
        out_specs=pl.BlockSpec(memory_space=pltpu.VMEM),
        scratch_shapes=[
            pltpu.VMEM((PN - 1, slice_r, half), x.dtype),
            pltpu.VMEM((PN - 1, slice_r, half), x.dtype),
            pltpu.VMEM((ZN - 1, chunk_r, half), x.dtype),
            pltpu.VMEM((ZN - 1, chunk_r, half), x.dtype),
            pltpu.VMEM((ZN - 1, chunk_r, half), x.dtype),
            pltpu.VMEM((ZN - 1, chunk_r, half), x.dtype),
            pltpu.VMEM((PN - 1, slice_r, half), x.dtype),
            pltpu.VMEM((PN - 1, slice_r, half), x.dtype),
            pltpu.SemaphoreType.DMA((PN - 1,)),
            pltpu.SemaphoreType.DMA((PN - 1,)),
            pltpu.SemaphoreType.DMA((PN - 1,)),
            pltpu.SemaphoreType.DMA((PN - 1,)),
            pltpu.SemaphoreType.DMA((ZN - 1,)),
            pltpu.SemaphoreType.DMA((ZN - 1,)),
            pltpu.SemaphoreType.DMA((ZN - 1,)),
            pltpu.SemaphoreType.DMA((ZN - 1,)),
            pltpu.SemaphoreType.DMA((ZN - 1,)),
            pltpu.SemaphoreType.DMA((ZN - 1,)),
            pltpu.SemaphoreType.DMA((ZN - 1,)),
            pltpu.SemaphoreType.DMA((ZN - 1,)),
            pltpu.SemaphoreType.DMA((PN - 1,)),
            pltpu.SemaphoreType.DMA((PN - 1,)),
            pltpu.SemaphoreType.DMA((PN - 1,)),
            pltpu.SemaphoreType.DMA((PN - 1,)),
        ],
        compiler_params=pltpu.CompilerParams(collective_id=0),
    )(x)
